# baseline (speedup 1.0000x reference)
import functools

import jax
import jax.numpy as jnp
from jax.experimental import pallas as pl
from jax.experimental.pallas import tpu as pltpu

B, S, D, C = 4, 2048, 2048, 64
N = B * S
EPS_LN = 1e-5
EPS_AGG = 1e-8

T = 256
DB = 512
NTB = N // T
NDB = D // DB
TPB = S // T
NC = 2
NTB2 = NTB // NC


def _pass1(x_ref, wq_ref, wk_ref, wr_ref, br_ref, g_ref, b_ref,
           rw_ref, z_ref, cnt_ref, nrm_ref,
           xn_s, logit_s, nq_s, nk_s):
    p = pl.program_id(0)
    tb_in = pl.program_id(1)
    tb = p * NTB2 + tb_in
    db = pl.program_id(2)

    @pl.when(db == 0)
    def _init():
        xx = x_ref[...]
        mu = jnp.mean(xx, axis=1, keepdims=True)
        xc = xx - mu
        var = jnp.mean(xc * xc, axis=1, keepdims=True)
        xn_s[...] = xc * jax.lax.rsqrt(var + EPS_LN) * g_ref[...] + b_ref[...]
        logit_s[...] = jnp.zeros_like(logit_s)
        nq_s[...] = jnp.zeros_like(nq_s)
        nk_s[...] = jnp.zeros_like(nk_s)

    xn = xn_s[...]
    qb = jax.lax.dot_general(xn, wq_ref[...], (((1,), (1,)), ((), ())),
                             preferred_element_type=jnp.float32)
    kb = jax.lax.dot_general(xn, wk_ref[...], (((1,), (1,)), ((), ())),
                             preferred_element_type=jnp.float32)
    nq_s[...] = nq_s[...] + jnp.sum(qb * qb, axis=1, keepdims=True)
    nk_s[...] = nk_s[...] + jnp.sum(kb * kb, axis=1, keepdims=True)
    rq = qb + 0.1 * kb
    logit_s[...] = logit_s[...] + jax.lax.dot_general(
        rq, wr_ref[...], (((1,), (1,)), ((), ())),
        preferred_element_type=jnp.float32)

    @pl.when(db == NDB - 1)
    def _finish():
        lg = logit_s[...] + br_ref[...]
        m = jnp.max(lg, axis=1, keepdims=True)
        e = jnp.exp(lg - m)
        rw = e / jnp.sum(e, axis=1, keepdims=True)
        rw_ref[...] = rw
        zc = jax.lax.dot_general(rw, xn, (((0,), (0,)), ((), ())),
                                 preferred_element_type=jnp.float32)
        cc = jnp.broadcast_to(jnp.sum(rw, axis=0, keepdims=True).T, (C, 128))
        nc = jnp.sum(jnp.sqrt(nq_s[...][:, :1]) + jnp.sqrt(nk_s[...][:, :1]),
                     axis=0, keepdims=True)
        nc = jnp.broadcast_to(nc, (1, 128))
        tb_loc = jax.lax.rem(tb, TPB)

        @pl.when(tb_loc == 0)
        def _():
            z_ref[0] = zc
            cnt_ref[0] = cc

        @pl.when(tb_loc != 0)
        def _():
            z_ref[0] = z_ref[0] + zc
            cnt_ref[0] = cnt_ref[0] + cc

        @pl.when(tb_in == 0)
        def _():
            nrm_ref[0] = nc

        @pl.when(tb_in != 0)
        def _():
            nrm_ref[0] = nrm_ref[0] + nc


def _pass2a(z_ref, cnt_ref, wv_ref, sc_ref, bi_ref, tr_ref):
    sums = jax.lax.dot_general(z_ref[0], wv_ref[...], (((1,), (1,)), ((), ())),
                               preferred_element_type=jnp.float32)
    cnt = cnt_ref[0][:, :1]
    means = sums / (cnt + EPS_AGG)
    tr_ref[0] = means * sc_ref[...] + bi_ref[...]


def _pass2b(tr_ref, wo_ref, t2_ref):
    t2_ref[0] = jax.lax.dot_general(tr_ref[0], wo_ref[...],
                                    (((1,), (1,)), ((), ())),
                                    preferred_element_type=jnp.float32)


def _pass3(x_ref, rw_ref, t2_ref, bo_ref, nrm_ref, out_ref):
    reg = 0.001 * (nrm_ref[0, 0:1, 0:1] + nrm_ref[1, 0:1, 0:1]) * (1.0 / N)
    agg = jax.lax.dot_general(rw_ref[...], t2_ref[0], (((1,), (0,)), ((), ())),
                              preferred_element_type=jnp.float32)
    out_ref[...] = agg + bo_ref[...] + reg + x_ref[...]


@jax.jit
def kernel(x, Wq, Wk, Wv, Wo, bo, ln_g, ln_b, Wr, br, agg_scale, agg_bias):
    x2 = x.reshape(N, D)
    br2 = br.reshape(1, C)
    g2 = ln_g.reshape(1, D)
    b2 = ln_b.reshape(1, D)
    bo2 = bo.reshape(1, D)

    rw, z, cnt, nrm = pl.pallas_call(
        _pass1,
        grid=(NC, NTB2, NDB),
        in_specs=[
            pl.BlockSpec((T, D), lambda p, tb, db: (p * NTB2 + tb, 0)),
            pl.BlockSpec((DB, D), lambda p, tb, db: (db, 0)),
            pl.BlockSpec((DB, D), lambda p, tb, db: (db, 0)),
            pl.BlockSpec((C, DB), lambda p, tb, db: (0, db)),
            pl.BlockSpec((1, C), lambda p, tb, db: (0, 0)),
            pl.BlockSpec((1, D), lambda p, tb, db: (0, 0)),
            pl.BlockSpec((1, D), lambda p, tb, db: (0, 0)),
        ],
        out_specs=[
            pl.BlockSpec((T, C), lambda p, tb, db: (p * NTB2 + tb, 0)),
            pl.BlockSpec((1, C, D),
                         lambda p, tb, db: ((p * NTB2 + tb) // TPB, 0, 0)),
            pl.BlockSpec((1, C, 128),
                         lambda p, tb, db: ((p * NTB2 + tb) // TPB, 0, 0)),
            pl.BlockSpec((1, 1, 128), lambda p, tb, db: (p, 0, 0)),
        ],
        out_shape=[
            jax.ShapeDtypeStruct((N, C), jnp.float32),
            jax.ShapeDtypeStruct((B, C, D), jnp.float32),
            jax.ShapeDtypeStruct((B, C, 128), jnp.float32),
            jax.ShapeDtypeStruct((NC, 1, 128), jnp.float32),
        ],
        scratch_shapes=[
            pltpu.VMEM((T, D), jnp.float32),
            pltpu.VMEM((T, C), jnp.float32),
            pltpu.VMEM((T, 128), jnp.float32),
            pltpu.VMEM((T, 128), jnp.float32),
        ],
        compiler_params=pltpu.CompilerParams(
            dimension_semantics=("parallel", "arbitrary", "arbitrary")),
    )(x2, Wq, Wk, Wr, br2, g2, b2)

    tr = pl.pallas_call(
        _pass2a,
        grid=(B, NDB),
        in_specs=[
            pl.BlockSpec((1, C, D), lambda b, db: (b, 0, 0)),
            pl.BlockSpec((1, C, 128), lambda b, db: (b, 0, 0)),
            pl.BlockSpec((DB, D), lambda b, db: (db, 0)),
            pl.BlockSpec((C, DB), lambda b, db: (0, db)),
            pl.BlockSpec((C, DB), lambda b, db: (0, db)),
        ],
        out_specs=pl.BlockSpec((1, C, DB), lambda b, db: (b, 0, db)),
        out_shape=jax.ShapeDtypeStruct((B, C, D), jnp.float32),
        compiler_params=pltpu.CompilerParams(
            dimension_semantics=("parallel", "arbitrary")),
    )(z, cnt, Wv, agg_scale, agg_bias)

    t2 = pl.pallas_call(
        _pass2b,
        grid=(B, NDB),
        in_specs=[
            pl.BlockSpec((1, C, D), lambda b, db: (b, 0, 0)),
            pl.BlockSpec((DB, D), lambda b, db: (db, 0)),
        ],
        out_specs=pl.BlockSpec((1, C, DB), lambda b, db: (b, 0, db)),
        out_shape=jax.ShapeDtypeStruct((B, C, D), jnp.float32),
        compiler_params=pltpu.CompilerParams(
            dimension_semantics=("parallel", "arbitrary")),
    )(tr, Wo)

    out = pl.pallas_call(
        _pass3,
        grid=(NTB,),
        in_specs=[
            pl.BlockSpec((T, D), lambda tb: (tb, 0)),
            pl.BlockSpec((T, C), lambda tb: (tb, 0)),
            pl.BlockSpec((1, C, D), lambda tb: (tb // TPB, 0, 0)),
            pl.BlockSpec((1, D), lambda tb: (0, 0)),
            pl.BlockSpec((NC, 1, 128), lambda tb: (0, 0, 0)),
        ],
        out_specs=pl.BlockSpec((T, D), lambda tb: (tb, 0)),
        out_shape=jax.ShapeDtypeStruct((N, D), jnp.float32),
        compiler_params=pltpu.CompilerParams(
            dimension_semantics=("parallel",)),
    )(x2, rw, t2, bo2, nrm)

    return out.reshape(B, S, D)


# speedup vs baseline: 1.4740x; 1.4740x over previous
import jax
import jax.numpy as jnp
from jax.experimental import pallas as pl
from jax.experimental.pallas import tpu as pltpu

B, S, D, C = 4, 2048, 2048, 64
N = B * S
EPS_LN = 1e-5
EPS_AGG = 1e-8

T = 512
DB = 512
NTB = N // T
NDB = D // DB
TPB = S // T
T3 = 256
NTB3 = N // T3
TPB3 = S // T3


def _pass1(x_ref, wq_ref, wk_ref, wr_ref, br_ref, g_ref, b_ref,
           rw_ref, z_ref, cnt_ref, nrm_ref,
           xn_s, logit_s, nq_s, nk_s):
    tb = pl.program_id(0)
    db = pl.program_id(1)

    @pl.when(db == 0)
    def _init():
        xx = x_ref[...]
        mu = jnp.mean(xx, axis=1, keepdims=True)
        xc = xx - mu
        var = jnp.mean(xc * xc, axis=1, keepdims=True)
        xn = xc * jax.lax.rsqrt(var + EPS_LN) * g_ref[...] + b_ref[...]
        xn_s[...] = xn.astype(jnp.bfloat16)
        logit_s[...] = jnp.zeros_like(logit_s)
        nq_s[...] = jnp.zeros_like(nq_s)
        nk_s[...] = jnp.zeros_like(nk_s)

    xn = xn_s[...]
    qb = jax.lax.dot_general(xn, wq_ref[...], (((1,), (1,)), ((), ())),
                             preferred_element_type=jnp.float32)
    kb = jax.lax.dot_general(xn, wk_ref[...], (((1,), (1,)), ((), ())),
                             preferred_element_type=jnp.float32)
    nq_s[...] = nq_s[...] + jnp.sum(qb * qb, axis=1, keepdims=True)
    nk_s[...] = nk_s[...] + jnp.sum(kb * kb, axis=1, keepdims=True)
    rq = (qb + 0.1 * kb).astype(jnp.bfloat16)
    logit_s[...] = logit_s[...] + jax.lax.dot_general(
        rq, wr_ref[...], (((1,), (1,)), ((), ())),
        preferred_element_type=jnp.float32)

    @pl.when(db == NDB - 1)
    def _finish():
        lg = logit_s[...] + br_ref[...]
        m = jnp.max(lg, axis=1, keepdims=True)
        e = jnp.exp(lg - m)
        rw = e / jnp.sum(e, axis=1, keepdims=True)
        rw_ref[...] = rw
        zc = jax.lax.dot_general(rw.astype(jnp.bfloat16), xn,
                                 (((0,), (0,)), ((), ())),
                                 preferred_element_type=jnp.float32)
        cc = jnp.broadcast_to(jnp.sum(rw, axis=0, keepdims=True).T, (C, 128))
        nc = jnp.sum(jnp.sqrt(nq_s[...][:, :1]) + jnp.sqrt(nk_s[...][:, :1]),
                     axis=0, keepdims=True)
        nc = jnp.broadcast_to(nc, (1, 128))
        tb_loc = jax.lax.rem(tb, TPB)

        @pl.when(tb_loc == 0)
        def _():
            z_ref[0] = zc
            cnt_ref[0] = cc

        @pl.when(tb_loc != 0)
        def _():
            z_ref[0] = z_ref[0] + zc
            cnt_ref[0] = cnt_ref[0] + cc

        @pl.when(tb == 0)
        def _():
            nrm_ref[0] = nc

        @pl.when(tb != 0)
        def _():
            nrm_ref[0] = nrm_ref[0] + nc


def _pass2(z_ref, cnt_ref, wv_ref, wo_ref, sc_ref, bi_ref, t2_ref):
    sums = jax.lax.dot_general(z_ref[0].astype(jnp.bfloat16), wv_ref[...],
                               (((1,), (1,)), ((), ())),
                               preferred_element_type=jnp.float32)
    cnt = cnt_ref[0][:, :1]
    means = sums / (cnt + EPS_AGG)
    tr = (means * sc_ref[...] + bi_ref[...]).astype(jnp.bfloat16)
    t2_ref[0] = jax.lax.dot_general(tr, wo_ref[...],
                                    (((1,), (1,)), ((), ())),
                                    preferred_element_type=jnp.float32)


def _pass3(x_ref, rw_ref, t2_ref, bo_ref, nrm_ref, out_ref):
    reg = 0.001 * nrm_ref[0, 0:1, 0:1] * (1.0 / N)
    agg = jax.lax.dot_general(rw_ref[...], t2_ref[0], (((1,), (0,)), ((), ())),
                              preferred_element_type=jnp.float32)
    out_ref[...] = agg + bo_ref[...] + reg + x_ref[...]


@jax.jit
def kernel(x, Wq, Wk, Wv, Wo, bo, ln_g, ln_b, Wr, br, agg_scale, agg_bias):
    x2 = x.reshape(N, D)
    br2 = br.reshape(1, C)
    g2 = ln_g.reshape(1, D)
    b2 = ln_b.reshape(1, D)
    bo2 = bo.reshape(1, D)
    wq16 = Wq.astype(jnp.bfloat16)
    wk16 = Wk.astype(jnp.bfloat16)
    wr16 = Wr.astype(jnp.bfloat16)
    wv16 = Wv.astype(jnp.bfloat16)
    wo16 = Wo.astype(jnp.bfloat16)

    rw, z, cnt, nrm = pl.pallas_call(
        _pass1,
        grid=(NTB, NDB),
        in_specs=[
            pl.BlockSpec((T, D), lambda tb, db: (tb, 0)),
            pl.BlockSpec((DB, D), lambda tb, db: (db, 0)),
            pl.BlockSpec((DB, D), lambda tb, db: (db, 0)),
            pl.BlockSpec((C, DB), lambda tb, db: (0, db)),
            pl.BlockSpec((1, C), lambda tb, db: (0, 0)),
            pl.BlockSpec((1, D), lambda tb, db: (0, 0)),
            pl.BlockSpec((1, D), lambda tb, db: (0, 0)),
        ],
        out_specs=[
            pl.BlockSpec((T, C), lambda tb, db: (tb, 0)),
            pl.BlockSpec((1, C, D), lambda tb, db: (tb // TPB, 0, 0)),
            pl.BlockSpec((1, C, 128), lambda tb, db: (tb // TPB, 0, 0)),
            pl.BlockSpec((1, 1, 128), lambda tb, db: (0, 0, 0)),
        ],
        out_shape=[
            jax.ShapeDtypeStruct((N, C), jnp.float32),
            jax.ShapeDtypeStruct((B, C, D), jnp.float32),
            jax.ShapeDtypeStruct((B, C, 128), jnp.float32),
            jax.ShapeDtypeStruct((1, 1, 128), jnp.float32),
        ],
        scratch_shapes=[
            pltpu.VMEM((T, D), jnp.bfloat16),
            pltpu.VMEM((T, C), jnp.float32),
            pltpu.VMEM((T, 128), jnp.float32),
            pltpu.VMEM((T, 128), jnp.float32),
        ],
        compiler_params=pltpu.CompilerParams(
            dimension_semantics=("arbitrary", "arbitrary")),
    )(x2, wq16, wk16, wr16, br2, g2, b2)

    t2 = pl.pallas_call(
        _pass2,
        grid=(B,),
        in_specs=[
            pl.BlockSpec((1, C, D), lambda b: (b, 0, 0)),
            pl.BlockSpec((1, C, 128), lambda b: (b, 0, 0)),
            pl.BlockSpec((D, D), lambda b: (0, 0)),
            pl.BlockSpec((D, D), lambda b: (0, 0)),
            pl.BlockSpec((C, D), lambda b: (0, 0)),
            pl.BlockSpec((C, D), lambda b: (0, 0)),
        ],
        out_specs=pl.BlockSpec((1, C, D), lambda b: (b, 0, 0)),
        out_shape=jax.ShapeDtypeStruct((B, C, D), jnp.float32),
        compiler_params=pltpu.CompilerParams(
            dimension_semantics=("arbitrary",)),
    )(z, cnt, wv16, wo16, agg_scale, agg_bias)

    out = pl.pallas_call(
        _pass3,
        grid=(NTB3,),
        in_specs=[
            pl.BlockSpec((T3, D), lambda tb: (tb, 0)),
            pl.BlockSpec((T3, C), lambda tb: (tb, 0)),
            pl.BlockSpec((1, C, D), lambda tb: (tb // TPB3, 0, 0)),
            pl.BlockSpec((1, D), lambda tb: (0, 0)),
            pl.BlockSpec((1, 1, 128), lambda tb: (0, 0, 0)),
        ],
        out_specs=pl.BlockSpec((T3, D), lambda tb: (tb, 0)),
        out_shape=jax.ShapeDtypeStruct((N, D), jnp.float32),
        compiler_params=pltpu.CompilerParams(
            dimension_semantics=("arbitrary",)),
    )(x2, rw, t2, bo2, nrm)

    return out.reshape(B, S, D)


# speedup vs baseline: 1.7233x; 1.1691x over previous
import jax
import jax.numpy as jnp
from jax.experimental import pallas as pl
from jax.experimental.pallas import tpu as pltpu

B, S, D, C = 4, 2048, 2048, 64
N = B * S
BC = B * C
EPS_LN = 1e-5
EPS_AGG = 1e-8

T = 512
NTB = N // T
TPB = S // T
DB2 = 512
NDB2 = D // DB2
T3 = 512
NTB3 = N // T3
TPB3 = S // T3


def _pass1(x_ref, wqk_ref, wr_ref, br_ref, g_ref, b_ref,
           rw_ref, z_ref, cnt_ref, nrm_ref):
    tb = pl.program_id(0)

    xx = x_ref[...]
    mu = jnp.mean(xx, axis=1, keepdims=True)
    xc = xx - mu
    var = jnp.mean(xc * xc, axis=1, keepdims=True)
    xn = xc * jax.lax.rsqrt(var + EPS_LN) * g_ref[...] + b_ref[...]
    xn16 = xn.astype(jnp.bfloat16)

    qkb = jax.lax.dot_general(xn16, wqk_ref[...], (((1,), (1,)), ((), ())),
                              preferred_element_type=jnp.float32)
    qb = qkb[:, :D]
    kb = qkb[:, D:]
    nq = jnp.sum(qb * qb, axis=1, keepdims=True)
    nk = jnp.sum(kb * kb, axis=1, keepdims=True)
    rq16 = (qb + 0.1 * kb).astype(jnp.bfloat16)
    lg = jax.lax.dot_general(rq16, wr_ref[...], (((1,), (1,)), ((), ())),
                             preferred_element_type=jnp.float32)
    lg = lg + br_ref[...]
    m = jnp.max(lg, axis=1, keepdims=True)
    e = jnp.exp(lg - m)
    rw = e / jnp.sum(e, axis=1, keepdims=True)
    rw_ref[...] = rw
    zc = jax.lax.dot_general(rw.astype(jnp.bfloat16), xn16,
                             (((0,), (0,)), ((), ())),
                             preferred_element_type=jnp.float32)
    cc = jnp.broadcast_to(jnp.sum(rw, axis=0, keepdims=True).T, (C, 128))
    nc = jnp.broadcast_to(
        jnp.sum(jnp.sqrt(nq) + jnp.sqrt(nk), axis=0, keepdims=True), (1, 128))
    tb_loc = jax.lax.rem(tb, TPB)

    @pl.when(tb_loc == 0)
    def _():
        z_ref[0] = zc
        cnt_ref[0] = cc

    @pl.when(tb_loc != 0)
    def _():
        z_ref[0] = z_ref[0] + zc
        cnt_ref[0] = cnt_ref[0] + cc

    @pl.when(tb == 0)
    def _():
        nrm_ref[0] = nc

    @pl.when(tb != 0)
    def _():
        nrm_ref[0] = nrm_ref[0] + nc


def _pass2a(z_ref, cnt_ref, wv_ref, sc_ref, bi_ref, tr_ref):
    wv16 = wv_ref[...].astype(jnp.bfloat16)
    z16 = z_ref[...].astype(jnp.bfloat16)
    sums = jax.lax.dot_general(z16, wv16, (((1,), (1,)), ((), ())),
                               preferred_element_type=jnp.float32)
    cnt = cnt_ref[...][:, :1]
    means = sums / (cnt + EPS_AGG)
    tr_ref[...] = (means * sc_ref[...] + bi_ref[...]).astype(jnp.bfloat16)


def _pass2b(tr_ref, wo_ref, t2_ref):
    wo16 = wo_ref[...].astype(jnp.bfloat16)
    t2_ref[...] = jax.lax.dot_general(tr_ref[...], wo16,
                                      (((1,), (1,)), ((), ())),
                                      preferred_element_type=jnp.float32)


def _pass3(x_ref, rw_ref, t2_ref, bo_ref, nrm_ref, out_ref):
    reg = 0.001 * nrm_ref[0, 0:1, 0:1] * (1.0 / N)
    agg = jax.lax.dot_general(rw_ref[...], t2_ref[...],
                              (((1,), (0,)), ((), ())),
                              preferred_element_type=jnp.float32)
    out_ref[...] = agg + bo_ref[...] + reg + x_ref[...]


@jax.jit
def kernel(x, Wq, Wk, Wv, Wo, bo, ln_g, ln_b, Wr, br, agg_scale, agg_bias):
    x2 = x.reshape(N, D)
    br2 = br.reshape(1, C)
    g2 = ln_g.reshape(1, D)
    b2 = ln_b.reshape(1, D)
    bo2 = bo.reshape(1, D)
    wqk16 = jnp.concatenate([Wq, Wk], axis=0).astype(jnp.bfloat16)
    wr16 = Wr.astype(jnp.bfloat16)
    sc4 = jnp.tile(agg_scale, (B, 1))
    bi4 = jnp.tile(agg_bias, (B, 1))

    rw, z, cnt, nrm = pl.pallas_call(
        _pass1,
        grid=(NTB,),
        in_specs=[
            pl.BlockSpec((T, D), lambda tb: (tb, 0)),
            pl.BlockSpec((2 * D, D), lambda tb: (0, 0)),
            pl.BlockSpec((C, D), lambda tb: (0, 0)),
            pl.BlockSpec((1, C), lambda tb: (0, 0)),
            pl.BlockSpec((1, D), lambda tb: (0, 0)),
            pl.BlockSpec((1, D), lambda tb: (0, 0)),
        ],
        out_specs=[
            pl.BlockSpec((T, C), lambda tb: (tb, 0)),
            pl.BlockSpec((1, C, D), lambda tb: (tb // TPB, 0, 0)),
            pl.BlockSpec((1, C, 128), lambda tb: (tb // TPB, 0, 0)),
            pl.BlockSpec((1, 1, 128), lambda tb: (0, 0, 0)),
        ],
        out_shape=[
            jax.ShapeDtypeStruct((N, C), jnp.float32),
            jax.ShapeDtypeStruct((B, C, D), jnp.float32),
            jax.ShapeDtypeStruct((B, C, 128), jnp.float32),
            jax.ShapeDtypeStruct((1, 1, 128), jnp.float32),
        ],
        compiler_params=pltpu.CompilerParams(
            dimension_semantics=("arbitrary",)),
    )(x2, wqk16, wr16, br2, g2, b2)

    z2 = z.reshape(BC, D)
    cnt2 = cnt.reshape(BC, 128)

    tr = pl.pallas_call(
        _pass2a,
        grid=(NDB2,),
        in_specs=[
            pl.BlockSpec((BC, D), lambda db: (0, 0)),
            pl.BlockSpec((BC, 128), lambda db: (0, 0)),
            pl.BlockSpec((DB2, D), lambda db: (db, 0)),
            pl.BlockSpec((BC, DB2), lambda db: (0, db)),
            pl.BlockSpec((BC, DB2), lambda db: (0, db)),
        ],
        out_specs=pl.BlockSpec((BC, DB2), lambda db: (0, db)),
        out_shape=jax.ShapeDtypeStruct((BC, D), jnp.bfloat16),
        compiler_params=pltpu.CompilerParams(
            dimension_semantics=("arbitrary",)),
    )(z2, cnt2, Wv, sc4, bi4)

    t2 = pl.pallas_call(
        _pass2b,
        grid=(NDB2,),
        in_specs=[
            pl.BlockSpec((BC, D), lambda db: (0, 0)),
            pl.BlockSpec((DB2, D), lambda db: (db, 0)),
        ],
        out_specs=pl.BlockSpec((BC, DB2), lambda db: (0, db)),
        out_shape=jax.ShapeDtypeStruct((BC, D), jnp.float32),
        compiler_params=pltpu.CompilerParams(
            dimension_semantics=("arbitrary",)),
    )(tr, Wo)

    out = pl.pallas_call(
        _pass3,
        grid=(NTB3,),
        in_specs=[
            pl.BlockSpec((T3, D), lambda tb: (tb, 0)),
            pl.BlockSpec((T3, C), lambda tb: (tb, 0)),
            pl.BlockSpec((C, D), lambda tb: (tb // TPB3, 0)),
            pl.BlockSpec((1, D), lambda tb: (0, 0)),
            pl.BlockSpec((1, 1, 128), lambda tb: (0, 0, 0)),
        ],
        out_specs=pl.BlockSpec((T3, D), lambda tb: (tb, 0)),
        out_shape=jax.ShapeDtypeStruct((N, D), jnp.float32),
        compiler_params=pltpu.CompilerParams(
            dimension_semantics=("arbitrary",)),
    )(x2, rw, t2, bo2, nrm)

    return out.reshape(B, S, D)
